# baseline (speedup 1.0000x reference)
import functools

import jax
import jax.numpy as jnp
from jax import lax
from jax.experimental import pallas as pl
from jax.experimental.pallas import tpu as pltpu
from jax.experimental.pallas import tpu_sc as plsc

_NUM_SLICES = 16
_SLICE_W = 4096 // _NUM_SLICES

_PERM = (7, 4, 2, 5, 3, 6, 10, 11, 15, 8, 9, 13, 14, 0, 1, 12)

_NC, _NS = 2, 16
_NW = _NC * _NS
_ROWS = 2 * 4096
_RPW = _ROWS // _NW

_MESH = plsc.VectorSubcoreMesh(
    core_axis_name="c", subcore_axis_name="s", num_cores=_NC, num_subcores=_NS
)


@functools.partial(
    pl.kernel,
    out_type=jax.ShapeDtypeStruct((_ROWS, _NUM_SLICES, _SLICE_W), jnp.float32),
    mesh=_MESH,
    scratch_types=[pltpu.SemaphoreType.DMA],
)
def _shuffle(in_hbm, out_hbm, sem):
    wid = lax.axis_index("s") * _NC + lax.axis_index("c")
    base = wid * _RPW
    copies = []
    for j in range(_NUM_SLICES):
        c = pltpu.make_async_copy(
            in_hbm.at[pl.ds(base, _RPW), pl.ds(_PERM[j], 1)],
            out_hbm.at[pl.ds(base, _RPW), pl.ds(j, 1)],
            sem,
        )
        c.start()
        copies.append(c)
    for c in copies:
        c.wait()


def kernel(x):
    shape = x.shape
    x3 = x.reshape(_ROWS, _NUM_SLICES, _SLICE_W)
    out = _shuffle(x3)
    return out.reshape(shape)


# speedup vs baseline: 11.4580x; 11.4580x over previous
import functools

import jax
import jax.numpy as jnp
from jax import lax
from jax.experimental import pallas as pl
from jax.experimental.pallas import tpu as pltpu
from jax.experimental.pallas import tpu_sc as plsc

_NUM_SLICES = 16
_SLICE_W = 256

_PERM = (7, 4, 2, 5, 3, 6, 10, 11, 15, 8, 9, 13, 14, 0, 1, 12)

_NC, _NS = 2, 16
_NW = _NC * _NS
_ROWS = 2 * 4096
_RPW = _ROWS // _NW

_R = 8
_CHUNKS = _RPW // _R

_MESH = plsc.VectorSubcoreMesh(
    core_axis_name="c", subcore_axis_name="s", num_cores=_NC, num_subcores=_NS
)


@functools.partial(
    pl.kernel,
    out_type=jax.ShapeDtypeStruct((_ROWS, _NUM_SLICES, _SLICE_W), jnp.float32),
    mesh=_MESH,
    scratch_types=[
        pltpu.VMEM((_R, _NUM_SLICES, _SLICE_W), jnp.float32),
        pltpu.VMEM((_R, _NUM_SLICES, _SLICE_W), jnp.float32),
        pltpu.SemaphoreType.DMA,
        pltpu.SemaphoreType.DMA,
        pltpu.SemaphoreType.DMA,
        pltpu.SemaphoreType.DMA,
    ],
)
def _shuffle(in_hbm, out_hbm, buf0, buf1, si0, si1, so0, so1):
    wid = lax.axis_index("s") * _NC + lax.axis_index("c")
    base = wid * _RPW

    def in_copies(c, buf, sem):
        row = base + c * _R
        return [
            pltpu.make_async_copy(
                in_hbm.at[pl.ds(row, _R), pl.ds(_PERM[j], 1)],
                buf.at[:, pl.ds(j, 1)],
                sem,
            )
            for j in range(_NUM_SLICES)
        ]

    def out_copy(c, buf, sem):
        row = base + c * _R
        return pltpu.make_async_copy(buf, out_hbm.at[pl.ds(row, _R)], sem)

    for d in in_copies(0, buf0, si0):
        d.start()
    for d in in_copies(1, buf1, si1):
        d.start()

    @pl.loop(0, _CHUNKS - 2, step=2)
    def _pipe(c):
        for d in in_copies(c, buf0, si0):
            d.wait()
        out_copy(c, buf0, so0).start()
        for d in in_copies(c + 1, buf1, si1):
            d.wait()
        out_copy(c + 1, buf1, so1).start()
        out_copy(c, buf0, so0).wait()
        for d in in_copies(c + 2, buf0, si0):
            d.start()
        out_copy(c + 1, buf1, so1).wait()
        for d in in_copies(c + 3, buf1, si1):
            d.start()

    c = _CHUNKS - 2
    for d in in_copies(c, buf0, si0):
        d.wait()
    out_copy(c, buf0, so0).start()
    for d in in_copies(c + 1, buf1, si1):
        d.wait()
    out_copy(c + 1, buf1, so1).start()
    out_copy(c, buf0, so0).wait()
    out_copy(c + 1, buf1, so1).wait()


def kernel(x):
    shape = x.shape
    x3 = x.reshape(_ROWS, _NUM_SLICES, _SLICE_W)
    out = _shuffle(x3)
    return out.reshape(shape)


# speedup vs baseline: 11.4725x; 1.0013x over previous
import functools

import jax
import jax.numpy as jnp
from jax import lax
from jax.experimental import pallas as pl
from jax.experimental.pallas import tpu as pltpu
from jax.experimental.pallas import tpu_sc as plsc

_NUM_SLICES = 16
_SLICE_W = 256

_PERM = (7, 4, 2, 5, 3, 6, 10, 11, 15, 8, 9, 13, 14, 0, 1, 12)

def _runs(perm):
    runs, j = [], 0
    while j < len(perm):
        k = j + 1
        while k < len(perm) and perm[k] == perm[k - 1] + 1:
            k += 1
        runs.append((j, perm[j], k - j))
        j = k
    return tuple(runs)

_RUNS = _runs(_PERM)

_NC, _NS = 2, 16
_NW = _NC * _NS
_ROWS = 2 * 4096
_RPW = _ROWS // _NW

_R = 8
_CHUNKS = _RPW // _R

_MESH = plsc.VectorSubcoreMesh(
    core_axis_name="c", subcore_axis_name="s", num_cores=_NC, num_subcores=_NS
)


@functools.partial(
    pl.kernel,
    out_type=jax.ShapeDtypeStruct((_ROWS, _NUM_SLICES, _SLICE_W), jnp.float32),
    mesh=_MESH,
    scratch_types=[
        pltpu.VMEM((_R, _NUM_SLICES, _SLICE_W), jnp.float32),
        pltpu.VMEM((_R, _NUM_SLICES, _SLICE_W), jnp.float32),
        pltpu.SemaphoreType.DMA,
        pltpu.SemaphoreType.DMA,
        pltpu.SemaphoreType.DMA,
        pltpu.SemaphoreType.DMA,
    ],
)
def _shuffle(in_hbm, out_hbm, buf0, buf1, si0, si1, so0, so1):
    wid = lax.axis_index("s") * _NC + lax.axis_index("c")
    base = wid * _RPW

    def in_copies(c, buf, sem):
        row = base + c * _R
        return [
            pltpu.make_async_copy(
                in_hbm.at[pl.ds(row, _R), pl.ds(src, ln)],
                buf.at[:, pl.ds(dst, ln)],
                sem,
            )
            for dst, src, ln in _RUNS
        ]

    def out_copy(c, buf, sem):
        row = base + c * _R
        return pltpu.make_async_copy(buf, out_hbm.at[pl.ds(row, _R)], sem)

    for d in in_copies(0, buf0, si0):
        d.start()
    for d in in_copies(1, buf1, si1):
        d.start()

    @pl.loop(0, _CHUNKS - 2, step=2)
    def _pipe(c):
        for d in in_copies(c, buf0, si0):
            d.wait()
        out_copy(c, buf0, so0).start()
        for d in in_copies(c + 1, buf1, si1):
            d.wait()
        out_copy(c + 1, buf1, so1).start()
        out_copy(c, buf0, so0).wait()
        for d in in_copies(c + 2, buf0, si0):
            d.start()
        out_copy(c + 1, buf1, so1).wait()
        for d in in_copies(c + 3, buf1, si1):
            d.start()

    c = _CHUNKS - 2
    for d in in_copies(c, buf0, si0):
        d.wait()
    out_copy(c, buf0, so0).start()
    for d in in_copies(c + 1, buf1, si1):
        d.wait()
    out_copy(c + 1, buf1, so1).start()
    out_copy(c, buf0, so0).wait()
    out_copy(c + 1, buf1, so1).wait()


def kernel(x):
    shape = x.shape
    x3 = x.reshape(_ROWS, _NUM_SLICES, _SLICE_W)
    out = _shuffle(x3)
    return out.reshape(shape)


# speedup vs baseline: 34.8221x; 3.0353x over previous
import functools

import jax
import jax.numpy as jnp
from jax import lax
from jax.experimental import pallas as pl
from jax.experimental.pallas import tpu as pltpu
from jax.experimental.pallas import tpu_sc as plsc

_NUM_SLICES = 16
_SLICE_W = 256

_PERM = (7, 4, 2, 5, 3, 6, 10, 11, 15, 8, 9, 13, 14, 0, 1, 12)


def _runs(perm):
    runs, j = [], 0
    while j < len(perm):
        k = j + 1
        while k < len(perm) and perm[k] == perm[k - 1] + 1:
            k += 1
        runs.append((j, perm[j], k - j))
        j = k
    return tuple(runs)


_RUNS = _runs(_PERM)

_NC, _NS = 2, 16
_NW = _NC * _NS
_ROWS = 2 * 4096
_COLS = 4096
_RPW = _ROWS // _NW

_R = 8
_CHUNKS = _RPW // _R

_MESH = plsc.VectorSubcoreMesh(
    core_axis_name="c", subcore_axis_name="s", num_cores=_NC, num_subcores=_NS
)


@functools.partial(
    pl.kernel,
    out_type=jax.ShapeDtypeStruct((_ROWS, _COLS), jnp.float32),
    mesh=_MESH,
    scratch_types=[
        pltpu.VMEM((_R, _COLS), jnp.float32),
        pltpu.VMEM((_R, _COLS), jnp.float32),
        pltpu.SemaphoreType.DMA,
        pltpu.SemaphoreType.DMA,
        pltpu.SemaphoreType.DMA,
        pltpu.SemaphoreType.DMA,
    ],
    compiler_params=pltpu.CompilerParams(use_tc_tiling_on_sc=True),
)
def _shuffle(in_hbm, out_hbm, buf0, buf1, si0, si1, so0, so1):
    wid = lax.axis_index("s") * _NC + lax.axis_index("c")
    base = wid * _RPW

    def in_copies(c, buf, sem):
        row = base + c * _R
        return [
            pltpu.make_async_copy(
                in_hbm.at[pl.ds(row, _R), pl.ds(src * _SLICE_W, ln * _SLICE_W)],
                buf.at[:, pl.ds(dst * _SLICE_W, ln * _SLICE_W)],
                sem,
            )
            for dst, src, ln in _RUNS
        ]

    def out_copy(c, buf, sem):
        row = base + c * _R
        return pltpu.make_async_copy(buf, out_hbm.at[pl.ds(row, _R)], sem)

    for d in in_copies(0, buf0, si0):
        d.start()
    for d in in_copies(1, buf1, si1):
        d.start()

    @pl.loop(0, _CHUNKS - 2, step=2)
    def _pipe(c):
        for d in in_copies(c, buf0, si0):
            d.wait()
        out_copy(c, buf0, so0).start()
        for d in in_copies(c + 1, buf1, si1):
            d.wait()
        out_copy(c + 1, buf1, so1).start()
        out_copy(c, buf0, so0).wait()
        for d in in_copies(c + 2, buf0, si0):
            d.start()
        out_copy(c + 1, buf1, so1).wait()
        for d in in_copies(c + 3, buf1, si1):
            d.start()

    c = _CHUNKS - 2
    for d in in_copies(c, buf0, si0):
        d.wait()
    out_copy(c, buf0, so0).start()
    for d in in_copies(c + 1, buf1, si1):
        d.wait()
    out_copy(c + 1, buf1, so1).start()
    out_copy(c, buf0, so0).wait()
    out_copy(c + 1, buf1, so1).wait()


def kernel(x):
    shape = x.shape
    x2 = x.reshape(_ROWS, _COLS)
    out = _shuffle(x2)
    return out.reshape(shape)


# speedup vs baseline: 35.0262x; 1.0059x over previous
import functools

import jax
import jax.numpy as jnp
from jax import lax
from jax.experimental import pallas as pl
from jax.experimental.pallas import tpu as pltpu
from jax.experimental.pallas import tpu_sc as plsc

_NUM_SLICES = 16
_SLICE_W = 256

_PERM = (7, 4, 2, 5, 3, 6, 10, 11, 15, 8, 9, 13, 14, 0, 1, 12)


def _runs(perm):
    runs, j = [], 0
    while j < len(perm):
        k = j + 1
        while k < len(perm) and perm[k] == perm[k - 1] + 1:
            k += 1
        runs.append((j, perm[j], k - j))
        j = k
    return tuple(runs)


_RUNS = _runs(_PERM)

_NC, _NS = 2, 16
_NW = _NC * _NS
_ROWS = 2 * 4096
_COLS = 4096
_RPW = _ROWS // _NW

_R = 8
_CHUNKS = _RPW // _R

_MESH = plsc.VectorSubcoreMesh(
    core_axis_name="c", subcore_axis_name="s", num_cores=_NC, num_subcores=_NS
)


@functools.partial(
    pl.kernel,
    out_type=jax.ShapeDtypeStruct((_ROWS, _COLS), jnp.float32),
    mesh=_MESH,
    scratch_types=[
        pltpu.VMEM((_R, _COLS), jnp.float32),
        pltpu.VMEM((_R, _COLS), jnp.float32),
        pltpu.SemaphoreType.DMA,
        pltpu.SemaphoreType.DMA,
        pltpu.SemaphoreType.DMA,
        pltpu.SemaphoreType.DMA,
    ],
    compiler_params=pltpu.CompilerParams(use_tc_tiling_on_sc=True),
)
def _shuffle(in_hbm, out_hbm, buf0, buf1, si0, si1, so0, so1):
    wid = lax.axis_index("s") * _NC + lax.axis_index("c")
    base = wid * _RPW

    def in_copies(c, buf, sem):
        row = base + c * _R
        return [
            pltpu.make_async_copy(
                in_hbm.at[pl.ds(row, _R), pl.ds(src * _SLICE_W, ln * _SLICE_W)],
                buf.at[:, pl.ds(dst * _SLICE_W, ln * _SLICE_W)],
                sem,
            )
            for dst, src, ln in _RUNS
        ]

    def out_copy(c, buf, sem):
        row = base + c * _R
        return pltpu.make_async_copy(buf, out_hbm.at[pl.ds(row, _R)], sem)

    def wait_in(c, buf, sem):
        row = base + c * _R
        pltpu.make_async_copy(in_hbm.at[pl.ds(row, _R)], buf, sem).wait()

    for d in in_copies(0, buf0, si0):
        d.start()
    for d in in_copies(1, buf1, si1):
        d.start()

    @pl.loop(0, _CHUNKS - 2, step=2)
    def _pipe(c):
        wait_in(c, buf0, si0)
        out_copy(c, buf0, so0).start()
        wait_in(c + 1, buf1, si1)
        out_copy(c + 1, buf1, so1).start()
        out_copy(c, buf0, so0).wait()
        for d in in_copies(c + 2, buf0, si0):
            d.start()
        out_copy(c + 1, buf1, so1).wait()
        for d in in_copies(c + 3, buf1, si1):
            d.start()

    c = _CHUNKS - 2
    wait_in(c, buf0, si0)
    out_copy(c, buf0, so0).start()
    wait_in(c + 1, buf1, si1)
    out_copy(c + 1, buf1, so1).start()
    out_copy(c, buf0, so0).wait()
    out_copy(c + 1, buf1, so1).wait()


def kernel(x):
    shape = x.shape
    x2 = x.reshape(_ROWS, _COLS)
    out = _shuffle(x2)
    return out.reshape(shape)


# speedup vs baseline: 36.0972x; 1.0306x over previous
import functools

import jax
import jax.numpy as jnp
from jax import lax
from jax.experimental import pallas as pl
from jax.experimental.pallas import tpu as pltpu
from jax.experimental.pallas import tpu_sc as plsc

_NUM_SLICES = 16
_SLICE_W = 256

_PERM = (7, 4, 2, 5, 3, 6, 10, 11, 15, 8, 9, 13, 14, 0, 1, 12)


def _runs(perm):
    runs, j = [], 0
    while j < len(perm):
        k = j + 1
        while k < len(perm) and perm[k] == perm[k - 1] + 1:
            k += 1
        runs.append((j, perm[j], k - j))
        j = k
    return tuple(runs)


_RUNS = _runs(_PERM)

_NC, _NS = 2, 16
_NW = _NC * _NS
_ROWS = 2 * 4096
_COLS = 4096
_RPW = _ROWS // _NW

_R = 8
_CHUNKS = _RPW // _R
_NBUF = 3

_MESH = plsc.VectorSubcoreMesh(
    core_axis_name="c", subcore_axis_name="s", num_cores=_NC, num_subcores=_NS
)


@functools.partial(
    pl.kernel,
    out_type=jax.ShapeDtypeStruct((_ROWS, _COLS), jnp.float32),
    mesh=_MESH,
    scratch_types=[
        pltpu.VMEM((_R, _COLS), jnp.float32),
        pltpu.VMEM((_R, _COLS), jnp.float32),
        pltpu.VMEM((_R, _COLS), jnp.float32),
        pltpu.SemaphoreType.DMA,
        pltpu.SemaphoreType.DMA,
        pltpu.SemaphoreType.DMA,
        pltpu.SemaphoreType.DMA,
        pltpu.SemaphoreType.DMA,
        pltpu.SemaphoreType.DMA,
    ],
    compiler_params=pltpu.CompilerParams(use_tc_tiling_on_sc=True),
)
def _shuffle(in_hbm, out_hbm, buf0, buf1, buf2, si0, si1, si2, so0, so1, so2):
    wid = lax.axis_index("s") * _NC + lax.axis_index("c")
    base = wid * _RPW
    bufs = (buf0, buf1, buf2)
    sis = (si0, si1, si2)
    sos = (so0, so1, so2)

    def start_in(c, b):
        row = base + c * _R
        for dst, src, ln in _RUNS:
            pltpu.make_async_copy(
                in_hbm.at[pl.ds(row, _R), pl.ds(src * _SLICE_W, ln * _SLICE_W)],
                bufs[b].at[:, pl.ds(dst * _SLICE_W, ln * _SLICE_W)],
                sis[b],
            ).start()

    def wait_in(c, b):
        row = base + c * _R
        pltpu.make_async_copy(in_hbm.at[pl.ds(row, _R)], bufs[b], sis[b]).wait()

    def out_copy(c, b):
        row = base + c * _R
        return pltpu.make_async_copy(bufs[b], out_hbm.at[pl.ds(row, _R)], sos[b])

    for b in range(_NBUF):
        start_in(b, b)
    wait_in(0, 0)
    out_copy(0, 0).start()

    @pl.loop(1, _CHUNKS - 4, step=_NBUF)
    def _pipe(p):
        for off in range(_NBUF):
            c = p + off
            b = (1 + off) % _NBUF
            bp = (0 + off) % _NBUF
            wait_in(c, b)
            out_copy(c, b).start()
            out_copy(c - 1, bp).wait()
            start_in(c + 2, bp)

    for c in range(_CHUNKS - 4, _CHUNKS):
        b = c % _NBUF
        bp = (c - 1) % _NBUF
        wait_in(c, b)
        out_copy(c, b).start()
        out_copy(c - 1, bp).wait()
        if c + 2 < _CHUNKS:
            start_in(c + 2, bp)
    out_copy(_CHUNKS - 1, (_CHUNKS - 1) % _NBUF).wait()


def kernel(x):
    shape = x.shape
    x2 = x.reshape(_ROWS, _COLS)
    out = _shuffle(x2)
    return out.reshape(shape)


# speedup vs baseline: 36.3176x; 1.0061x over previous
import functools

import jax
import jax.numpy as jnp
from jax import lax
from jax.experimental import pallas as pl
from jax.experimental.pallas import tpu as pltpu
from jax.experimental.pallas import tpu_sc as plsc

_NUM_SLICES = 16
_SLICE_W = 256

_PERM = (7, 4, 2, 5, 3, 6, 10, 11, 15, 8, 9, 13, 14, 0, 1, 12)


def _runs(perm):
    runs, j = [], 0
    while j < len(perm):
        k = j + 1
        while k < len(perm) and perm[k] == perm[k - 1] + 1:
            k += 1
        runs.append((j, perm[j], k - j))
        j = k
    return tuple(runs)


_RUNS = _runs(_PERM)

_NC, _NS = 2, 16
_NW = _NC * _NS
_ROWS = 2 * 4096
_COLS = 4096
_RPW = _ROWS // _NW

_R = 8
_CHUNKS = _RPW // _R
_NBUF = 3

_MESH = plsc.VectorSubcoreMesh(
    core_axis_name="c", subcore_axis_name="s", num_cores=_NC, num_subcores=_NS
)


@functools.partial(
    pl.kernel,
    out_type=jax.ShapeDtypeStruct((_ROWS, _COLS), jnp.float32),
    mesh=_MESH,
    scratch_types=[
        pltpu.VMEM((_R, _COLS), jnp.float32),
        pltpu.VMEM((_R, _COLS), jnp.float32),
        pltpu.VMEM((_R, _COLS), jnp.float32),
        pltpu.SemaphoreType.DMA,
        pltpu.SemaphoreType.DMA,
        pltpu.SemaphoreType.DMA,
        pltpu.SemaphoreType.DMA,
        pltpu.SemaphoreType.DMA,
        pltpu.SemaphoreType.DMA,
    ],
    compiler_params=pltpu.CompilerParams(use_tc_tiling_on_sc=True),
)
def _shuffle(in_hbm, out_hbm, buf0, buf1, buf2, si0, si1, si2, so0, so1, so2):
    wid = lax.axis_index("s") * _NC + lax.axis_index("c")
    base = wid * _RPW
    bufs = (buf0, buf1, buf2)
    sis = (si0, si1, si2)
    sos = (so0, so1, so2)

    def start_in(c, b):
        row = base + c * _R
        for dst, src, ln in _RUNS:
            pltpu.make_async_copy(
                in_hbm.at[pl.ds(row, _R), pl.ds(src * _SLICE_W, ln * _SLICE_W)],
                bufs[b].at[:, pl.ds(dst * _SLICE_W, ln * _SLICE_W)],
                sis[b],
            ).start()

    def wait_in(c, b):
        row = base + c * _R
        pltpu.make_async_copy(in_hbm.at[pl.ds(row, _R)], bufs[b], sis[b]).wait()

    def out_copy(c, b):
        row = base + c * _R
        return pltpu.make_async_copy(bufs[b], out_hbm.at[pl.ds(row, _R)], sos[b])

    for b in range(_NBUF):
        start_in(b, b)
    wait_in(0, 0)
    out_copy(0, 0).start()

    @pl.loop(1, _CHUNKS - 4, step=_NBUF)
    def _pipe(p):
        for off in range(_NBUF):
            c = p + off
            b = (1 + off) % _NBUF
            bp = (0 + off) % _NBUF
            out_copy(c - 1, bp).wait()
            start_in(c + 2, bp)
            wait_in(c, b)
            out_copy(c, b).start()

    for c in range(_CHUNKS - 4, _CHUNKS):
        b = c % _NBUF
        bp = (c - 1) % _NBUF
        out_copy(c - 1, bp).wait()
        if c + 2 < _CHUNKS:
            start_in(c + 2, bp)
        wait_in(c, b)
        out_copy(c, b).start()
    out_copy(_CHUNKS - 1, (_CHUNKS - 1) % _NBUF).wait()


def kernel(x):
    shape = x.shape
    x2 = x.reshape(_ROWS, _COLS)
    out = _shuffle(x2)
    return out.reshape(shape)
